# baseline (speedup 1.0000x reference)
import functools

import jax
import jax.numpy as jnp
from jax import lax
from jax.experimental import pallas as pl
from jax.experimental.pallas import tpu as pltpu
from jax.experimental.pallas import tpu_sc as plsc

BATCH = 16384
SEQ = 20
EMB_SIZE = 512
EMB_DIM = 4

_BB = 2048
_BIG = 1 << 20


def _dp_body(a_ref, b_ref, out_ref):
    a = a_ref[...]
    b = b_ref[...]
    bb = a.shape[1]
    arange = lax.broadcasted_iota(jnp.int32, (SEQ + 1, bb), 0)
    row = arange
    for i in range(SEQ):
        cost = (a[i:i + 1, :] != b).astype(jnp.int32)
        up = row[1:, :]
        diag = row[:SEQ, :]
        t = jnp.minimum(up + 1, diag + cost)
        t_full = jnp.concatenate(
            [jnp.full((1, bb), i + 1, jnp.int32), t], axis=0)
        u = t_full - arange
        for s in (1, 2, 4, 8, 16):
            shifted = jnp.concatenate(
                [jnp.full((s, bb), _BIG, jnp.int32), u[:SEQ + 1 - s, :]], axis=0)
            u = jnp.minimum(u, shifted)
        row = u + arange
    dist = row[SEQ, :]
    out_ref[...] = jnp.minimum(dist, EMB_SIZE - 1).reshape(1, bb)


def _edit_distance_ids(a_t, b_t):
    grid = BATCH // _BB
    out = pl.pallas_call(
        _dp_body,
        grid=(grid,),
        in_specs=[
            pl.BlockSpec((SEQ, _BB), lambda i: (0, i)),
            pl.BlockSpec((SEQ, _BB), lambda i: (0, i)),
        ],
        out_specs=pl.BlockSpec((1, _BB), lambda i: (0, i)),
        out_shape=jax.ShapeDtypeStruct((1, BATCH), jnp.int32),
    )(a_t, b_t)
    return out.reshape(BATCH)


_NC = 2
_NS = 16
_NW = _NC * _NS
_L = 16
_CHUNK = BATCH // _NW
_IDXW = 128
_DPAD = 128


@functools.cache
def _sc_gather_fn():
    mesh = plsc.VectorSubcoreMesh(
        core_axis_name="c", subcore_axis_name="s",
        num_cores=_NC, num_subcores=_NS)

    n_rows = _CHUNK // _IDXW

    @functools.partial(
        pl.kernel,
        mesh=mesh,
        out_type=jax.ShapeDtypeStruct((BATCH, _DPAD), jnp.float32),
        scratch_types=[
            pltpu.VMEM((n_rows, _IDXW), jnp.int32),
            pltpu.VMEM((_CHUNK, _DPAD), jnp.float32),
            pltpu.SemaphoreType.DMA,
        ],
        compiler_params=pltpu.CompilerParams(skip_device_barrier=True),
    )
    def _sc_gather(table_hbm, ids_hbm, out_hbm, idx_v, rows_v, sem):
        wid = lax.axis_index("s") * _NC + lax.axis_index("c")
        base = wid * _CHUNK
        pltpu.sync_copy(ids_hbm.at[pl.ds(wid * n_rows, n_rows)], idx_v)
        copies = [
            pltpu.async_copy(
                table_hbm.at[idx_v.at[j]],
                rows_v.at[pl.ds(j * _IDXW, _IDXW)], sem)
            for j in range(n_rows)
        ]
        for c in copies:
            c.wait()
        pltpu.sync_copy(rows_v, out_hbm.at[pl.ds(base, _CHUNK)])

    return _sc_gather


def kernel(input1, input2, embedding_table):
    ids = _edit_distance_ids(input1.T, input2.T)
    table_pad = jnp.pad(embedding_table, ((0, 0), (0, _DPAD - EMB_DIM)))
    ids2 = ids.reshape(BATCH // _IDXW, _IDXW)
    out_pad = _sc_gather_fn()(table_pad, ids2)
    return out_pad[:, :EMB_DIM]


# speedup vs baseline: 4.5045x; 4.5045x over previous
import functools

import jax
import jax.numpy as jnp
from jax import lax
from jax.experimental import pallas as pl
from jax.experimental.pallas import tpu as pltpu
from jax.experimental.pallas import tpu_sc as plsc

BATCH = 16384
SEQ = 20
EMB_SIZE = 512
EMB_DIM = 4

_BB = 2048
_BIG = 1 << 20


def _dp_body(a_ref, b_ref, out_ref):
    a = a_ref[...]
    b = b_ref[...]
    bb = a.shape[1]
    arange = lax.broadcasted_iota(jnp.int32, (SEQ + 1, bb), 0)
    row = arange
    for i in range(SEQ):
        cost = (a[i:i + 1, :] != b).astype(jnp.int32)
        up = row[1:, :]
        diag = row[:SEQ, :]
        t = jnp.minimum(up + 1, diag + cost)
        t_full = jnp.concatenate(
            [jnp.full((1, bb), i + 1, jnp.int32), t], axis=0)
        u = t_full - arange
        for s in (1, 2, 4, 8, 16):
            shifted = jnp.concatenate(
                [jnp.full((s, bb), _BIG, jnp.int32), u[:SEQ + 1 - s, :]], axis=0)
            u = jnp.minimum(u, shifted)
        row = u + arange
    dist = row[SEQ, :]
    out_ref[...] = jnp.minimum(dist, EMB_SIZE - 1).reshape(1, bb)


def _edit_distance_ids(a_t, b_t):
    grid = BATCH // _BB
    out = pl.pallas_call(
        _dp_body,
        grid=(grid,),
        in_specs=[
            pl.BlockSpec((SEQ, _BB), lambda i: (0, i)),
            pl.BlockSpec((SEQ, _BB), lambda i: (0, i)),
        ],
        out_specs=pl.BlockSpec((1, _BB), lambda i: (0, i)),
        out_shape=jax.ShapeDtypeStruct((1, BATCH), jnp.int32),
    )(a_t, b_t)
    return out.reshape(BATCH)


_NC = 2
_NS = 16
_NW = _NC * _NS
_L = 16
_CHUNK = BATCH // _NW
_IDXW = 128
_DPAD = 16


@functools.cache
def _sc_gather_fn():
    mesh = plsc.VectorSubcoreMesh(
        core_axis_name="c", subcore_axis_name="s",
        num_cores=_NC, num_subcores=_NS)

    n_rows = _CHUNK // _IDXW

    @functools.partial(
        pl.kernel,
        mesh=mesh,
        out_type=jax.ShapeDtypeStruct((BATCH, _DPAD), jnp.float32),
        scratch_types=[
            pltpu.VMEM((n_rows, _IDXW), jnp.int32),
            pltpu.VMEM((_CHUNK, _DPAD), jnp.float32),
            pltpu.SemaphoreType.DMA,
        ],
        compiler_params=pltpu.CompilerParams(use_tc_tiling_on_sc=False),
    )
    def _sc_gather(table_hbm, ids_hbm, out_hbm, idx_v, rows_v, sem):
        wid = lax.axis_index("s") * _NC + lax.axis_index("c")
        base = wid * _CHUNK
        pltpu.sync_copy(ids_hbm.at[pl.ds(wid * n_rows, n_rows)], idx_v)
        copies = [
            pltpu.async_copy(
                table_hbm.at[idx_v.at[j]],
                rows_v.at[pl.ds(j * _IDXW, _IDXW)], sem)
            for j in range(n_rows)
        ]
        for c in copies:
            c.wait()
        pltpu.sync_copy(rows_v, out_hbm.at[pl.ds(base, _CHUNK)])

    return _sc_gather


def kernel(input1, input2, embedding_table):
    ids = _edit_distance_ids(input1.T, input2.T)
    table_pad = jnp.pad(embedding_table, ((0, 0), (0, _DPAD - EMB_DIM)))
    ids2 = ids.reshape(BATCH // _IDXW, _IDXW)
    out_pad = _sc_gather_fn()(table_pad, ids2)
    return out_pad[:, :EMB_DIM]


# speedup vs baseline: 8.9599x; 1.9891x over previous
import functools

import jax
import jax.numpy as jnp
from jax import lax
from jax.experimental import pallas as pl
from jax.experimental.pallas import tpu as pltpu
from jax.experimental.pallas import tpu_sc as plsc

BATCH = 16384
SEQ = 20
EMB_SIZE = 512
EMB_DIM = 4

_BB = 2048
_BIG = 1 << 20


def _dp_body(a_ref, b_ref, out_ref):
    a = a_ref[...]
    b = b_ref[...]
    bb = a.shape[1]
    arange = lax.broadcasted_iota(jnp.int32, (SEQ + 1, bb), 0)
    row = arange
    for i in range(SEQ):
        cost = (a[i:i + 1, :] != b).astype(jnp.int32)
        up = row[1:, :]
        diag = row[:SEQ, :]
        t = jnp.minimum(up + 1, diag + cost)
        t_full = jnp.concatenate(
            [jnp.full((1, bb), i + 1, jnp.int32), t], axis=0)
        u = t_full - arange
        for s in (1, 2, 4, 8, 16):
            shifted = jnp.concatenate(
                [jnp.full((s, bb), _BIG, jnp.int32), u[:SEQ + 1 - s, :]], axis=0)
            u = jnp.minimum(u, shifted)
        row = u + arange
    dist = row[SEQ, :]
    out_ref[...] = jnp.minimum(dist, EMB_SIZE - 1).reshape(1, bb)


def _edit_distance_ids(a_t, b_t):
    grid = BATCH // _BB
    out = pl.pallas_call(
        _dp_body,
        grid=(grid,),
        in_specs=[
            pl.BlockSpec((SEQ, _BB), lambda i: (0, i)),
            pl.BlockSpec((SEQ, _BB), lambda i: (0, i)),
        ],
        out_specs=pl.BlockSpec((1, _BB), lambda i: (0, i)),
        out_shape=jax.ShapeDtypeStruct((1, BATCH), jnp.int32),
    )(a_t, b_t)
    return out.reshape(BATCH)


_NC = 2
_NS = 16
_NW = _NC * _NS
_L = 16
_CHUNK = BATCH // _NW
_IDXW = 128
_DPAD = 16


@functools.cache
def _sc_gather_fn():
    mesh = plsc.VectorSubcoreMesh(
        core_axis_name="c", subcore_axis_name="s",
        num_cores=_NC, num_subcores=_NS)

    @functools.partial(
        pl.kernel,
        mesh=mesh,
        out_type=jax.ShapeDtypeStruct((BATCH * EMB_DIM,), jnp.float32),
        scratch_types=[
            pltpu.VMEM((EMB_SIZE * EMB_DIM,), jnp.float32),
            pltpu.VMEM((_CHUNK,), jnp.int32),
            pltpu.VMEM((_CHUNK * EMB_DIM,), jnp.float32),
        ],
        compiler_params=pltpu.CompilerParams(
            use_tc_tiling_on_sc=False, needs_layout_passes=False),
    )
    def _sc_gather(table_hbm, ids_hbm, out_hbm, table_v, ids_v, out_v):
        wid = lax.axis_index("s") * _NC + lax.axis_index("c")
        base = wid * _CHUNK
        pltpu.sync_copy(table_hbm, table_v)
        pltpu.sync_copy(ids_hbm.at[pl.ds(base, _CHUNK)], ids_v)
        lanes = lax.iota(jnp.int32, _L)
        for g in range(_CHUNK // _L):
            ids16 = ids_v[pl.ds(g * _L, _L)]
            rowbase = ids16 * EMB_DIM
            pos0 = lanes * EMB_DIM + g * _L * EMB_DIM
            for c in range(EMB_DIM):
                vals = plsc.load_gather(table_v, [rowbase + c])
                plsc.store_scatter(out_v, [pos0 + c], vals)
        pltpu.sync_copy(
            out_v, out_hbm.at[pl.ds(base * EMB_DIM, _CHUNK * EMB_DIM)])

    return _sc_gather


def kernel(input1, input2, embedding_table):
    ids = _edit_distance_ids(input1.T, input2.T)
    out_flat = _sc_gather_fn()(embedding_table.reshape(-1), ids)
    return out_flat.reshape(BATCH, EMB_DIM)


# speedup vs baseline: 9.4390x; 1.0535x over previous
import functools

import jax
import jax.numpy as jnp
from jax import lax
from jax.experimental import pallas as pl
from jax.experimental.pallas import tpu as pltpu
from jax.experimental.pallas import tpu_sc as plsc

BATCH = 16384
SEQ = 20
EMB_SIZE = 512
EMB_DIM = 4

_BB = 2048
_BIG = 1 << 20


def _dp_body(a_ref, b_ref, out_ref):
    a = a_ref[...]
    b = b_ref[...]
    bb = a.shape[1]
    arange = lax.broadcasted_iota(jnp.int32, (SEQ + 1, bb), 0)
    row = arange
    for i in range(SEQ):
        cost = (a[i:i + 1, :] != b).astype(jnp.int32)
        up = row[1:, :]
        diag = row[:SEQ, :]
        t = jnp.minimum(up + 1, diag + cost)
        t_full = jnp.concatenate(
            [jnp.full((1, bb), i + 1, jnp.int32), t], axis=0)
        u = t_full - arange
        for s in (1, 2, 4, 8, 16):
            shifted = jnp.concatenate(
                [jnp.full((s, bb), _BIG, jnp.int32), u[:SEQ + 1 - s, :]], axis=0)
            u = jnp.minimum(u, shifted)
        row = u + arange
    dist = row[SEQ, :]
    out_ref[...] = jnp.minimum(dist, EMB_SIZE - 1).reshape(1, bb)


def _edit_distance_ids(a_t, b_t):
    grid = BATCH // _BB
    out = pl.pallas_call(
        _dp_body,
        grid=(grid,),
        in_specs=[
            pl.BlockSpec((SEQ, _BB), lambda i: (0, i)),
            pl.BlockSpec((SEQ, _BB), lambda i: (0, i)),
        ],
        out_specs=pl.BlockSpec((1, _BB), lambda i: (0, i)),
        out_shape=jax.ShapeDtypeStruct((1, BATCH), jnp.int32),
    )(a_t, b_t)
    return out.reshape(BATCH)


_NC = 2
_NS = 16
_NW = _NC * _NS
_L = 16
_CHUNK = BATCH // _NW
_IDXW = 128
_DPAD = 16


@functools.cache
def _sc_gather_fn():
    mesh = plsc.VectorSubcoreMesh(
        core_axis_name="c", subcore_axis_name="s",
        num_cores=_NC, num_subcores=_NS)

    @functools.partial(
        pl.kernel,
        mesh=mesh,
        out_type=jax.ShapeDtypeStruct((BATCH, EMB_DIM), jnp.float32),
        scratch_types=[
            pltpu.VMEM((EMB_SIZE * EMB_DIM,), jnp.float32),
            pltpu.VMEM((_CHUNK,), jnp.int32),
            pltpu.VMEM((_CHUNK, EMB_DIM), jnp.float32),
        ],
        compiler_params=pltpu.CompilerParams(
            use_tc_tiling_on_sc=False, needs_layout_passes=False),
    )
    def _sc_gather(table_hbm, ids_hbm, out_hbm, table_v, ids_v, out_v):
        wid = lax.axis_index("s") * _NC + lax.axis_index("c")
        base = wid * _CHUNK
        pltpu.sync_copy(table_hbm, table_v)
        pltpu.sync_copy(ids_hbm.at[pl.ds(base, _CHUNK)], ids_v)
        lanes = lax.iota(jnp.int32, _L)
        for g in range(_CHUNK // _L):
            ids16 = ids_v[pl.ds(g * _L, _L)]
            rowbase = ids16 * EMB_DIM
            rows = lanes + g * _L
            for c in range(EMB_DIM):
                vals = plsc.load_gather(table_v, [rowbase + c])
                cols = jnp.full((_L,), c, jnp.int32)
                plsc.store_scatter(out_v, [rows, cols], vals)
        pltpu.sync_copy(out_v, out_hbm.at[pl.ds(base, _CHUNK)])

    return _sc_gather


def kernel(input1, input2, embedding_table):
    ids = _edit_distance_ids(input1.T, input2.T)
    return _sc_gather_fn()(embedding_table.reshape(-1), ids)


# speedup vs baseline: 10.0915x; 1.0691x over previous
import functools

import jax
import jax.numpy as jnp
from jax import lax
from jax.experimental import pallas as pl
from jax.experimental.pallas import tpu as pltpu
from jax.experimental.pallas import tpu_sc as plsc

BATCH = 16384
SEQ = 20
EMB_SIZE = 512
EMB_DIM = 4

_BB = 2048
_BIG = 128.0


def _dp_body(a_ref, b_ref, out_ref):
    a = a_ref[...].astype(jnp.bfloat16)
    b = b_ref[...].astype(jnp.bfloat16)
    bb = a.shape[1]
    arange = lax.broadcasted_iota(
        jnp.int32, (SEQ + 1, bb), 0).astype(jnp.bfloat16)
    row = arange
    one = jnp.bfloat16(1)
    for i in range(SEQ):
        cost = (a[i:i + 1, :] != b).astype(jnp.bfloat16)
        up = row[1:, :]
        diag = row[:SEQ, :]
        t = jnp.minimum(up + one, diag + cost)
        t_full = jnp.concatenate(
            [jnp.full((1, bb), i + 1, jnp.bfloat16), t], axis=0)
        u = t_full - arange
        for s in (1, 2, 4, 8, 16):
            shifted = jnp.concatenate(
                [jnp.full((s, bb), _BIG, jnp.bfloat16), u[:SEQ + 1 - s, :]], axis=0)
            u = jnp.minimum(u, shifted)
        row = u + arange
    dist = row[SEQ, :].astype(jnp.int32)
    out_ref[...] = jnp.minimum(dist, EMB_SIZE - 1).reshape(1, bb)


def _edit_distance_ids(a_t, b_t):
    grid = BATCH // _BB
    out = pl.pallas_call(
        _dp_body,
        grid=(grid,),
        in_specs=[
            pl.BlockSpec((SEQ, _BB), lambda i: (0, i)),
            pl.BlockSpec((SEQ, _BB), lambda i: (0, i)),
        ],
        out_specs=pl.BlockSpec((1, _BB), lambda i: (0, i)),
        out_shape=jax.ShapeDtypeStruct((1, BATCH), jnp.int32),
    )(a_t, b_t)
    return out.reshape(BATCH)


_NC = 2
_NS = 16
_NW = _NC * _NS
_L = 16
_CHUNK = BATCH // _NW
_IDXW = 128
_DPAD = 16


@functools.cache
def _sc_gather_fn():
    mesh = plsc.VectorSubcoreMesh(
        core_axis_name="c", subcore_axis_name="s",
        num_cores=_NC, num_subcores=_NS)

    @functools.partial(
        pl.kernel,
        mesh=mesh,
        out_type=jax.ShapeDtypeStruct((BATCH, EMB_DIM), jnp.float32),
        scratch_types=[
            pltpu.VMEM((EMB_SIZE * EMB_DIM,), jnp.float32),
            pltpu.VMEM((_CHUNK,), jnp.int32),
            pltpu.VMEM((_CHUNK, EMB_DIM), jnp.float32),
        ],
        compiler_params=pltpu.CompilerParams(
            use_tc_tiling_on_sc=False, needs_layout_passes=False),
    )
    def _sc_gather(table_hbm, ids_hbm, out_hbm, table_v, ids_v, out_v):
        wid = lax.axis_index("s") * _NC + lax.axis_index("c")
        base = wid * _CHUNK
        pltpu.sync_copy(table_hbm, table_v)
        pltpu.sync_copy(ids_hbm.at[pl.ds(base, _CHUNK)], ids_v)
        lanes = lax.iota(jnp.int32, _L)
        for g in range(_CHUNK // _L):
            ids16 = ids_v[pl.ds(g * _L, _L)]
            rowbase = ids16 * EMB_DIM
            rows = lanes + g * _L
            for c in range(EMB_DIM):
                vals = plsc.load_gather(table_v, [rowbase + c])
                cols = jnp.full((_L,), c, jnp.int32)
                plsc.store_scatter(out_v, [rows, cols], vals)
        pltpu.sync_copy(out_v, out_hbm.at[pl.ds(base, _CHUNK)])

    return _sc_gather


def kernel(input1, input2, embedding_table):
    ids = _edit_distance_ids(input1.T, input2.T)
    return _sc_gather_fn()(embedding_table.reshape(-1), ids)


# speedup vs baseline: 11.4928x; 1.1389x over previous
import functools

import jax
import jax.numpy as jnp
from jax import lax
from jax.experimental import pallas as pl
from jax.experimental.pallas import tpu as pltpu
from jax.experimental.pallas import tpu_sc as plsc

BATCH = 16384
SEQ = 20
EMB_SIZE = 512
EMB_DIM = 4

_BB = 2048
_BIG = 128.0


def _dp_body(a_ref, b_ref, out_ref):
    a = a_ref[...].astype(jnp.bfloat16)
    b = b_ref[...].astype(jnp.bfloat16)
    bb = a.shape[1]
    one = jnp.bfloat16(1)
    r = jnp.zeros((SEQ + 1, bb), jnp.bfloat16)
    for i in range(SEQ):
        cm1 = jnp.where(a[i:i + 1, :] != b, jnp.bfloat16(0),
                        jnp.bfloat16(-1))
        u = jnp.minimum(
            r + one,
            jnp.concatenate([jnp.full((1, bb), i + 1, jnp.bfloat16),
                             r[:SEQ, :] + cm1], axis=0))
        for s in (1, 2, 4, 8, 16):
            shifted = jnp.concatenate(
                [jnp.full((s, bb), _BIG, jnp.bfloat16), u[:SEQ + 1 - s, :]],
                axis=0)
            u = jnp.minimum(u, shifted)
        r = u
    dist = r[SEQ, :].astype(jnp.int32) + SEQ
    out_ref[...] = jnp.minimum(dist, EMB_SIZE - 1).reshape(1, bb)


def _edit_distance_ids(a_t, b_t):
    grid = BATCH // _BB
    out = pl.pallas_call(
        _dp_body,
        grid=(grid,),
        in_specs=[
            pl.BlockSpec((SEQ, _BB), lambda i: (0, i)),
            pl.BlockSpec((SEQ, _BB), lambda i: (0, i)),
        ],
        out_specs=pl.BlockSpec((1, _BB), lambda i: (0, i)),
        out_shape=jax.ShapeDtypeStruct((1, BATCH), jnp.int32),
    )(a_t, b_t)
    return out.reshape(BATCH)


_NC = 2
_NS = 16
_NW = _NC * _NS
_L = 16
_CHUNK = BATCH // _NW
_IDXW = 128
_DPAD = 16


@functools.cache
def _sc_gather_fn():
    mesh = plsc.VectorSubcoreMesh(
        core_axis_name="c", subcore_axis_name="s",
        num_cores=_NC, num_subcores=_NS)

    @functools.partial(
        pl.kernel,
        mesh=mesh,
        out_type=jax.ShapeDtypeStruct((BATCH, EMB_DIM), jnp.float32),
        scratch_types=[
            pltpu.VMEM((EMB_SIZE * EMB_DIM,), jnp.float32),
            pltpu.VMEM((_CHUNK,), jnp.int32),
            pltpu.VMEM((_CHUNK, EMB_DIM), jnp.float32),
        ],
        compiler_params=pltpu.CompilerParams(
            use_tc_tiling_on_sc=False, needs_layout_passes=False),
    )
    def _sc_gather(table_hbm, ids_hbm, out_hbm, table_v, ids_v, out_v):
        wid = lax.axis_index("s") * _NC + lax.axis_index("c")
        base = wid * _CHUNK
        pltpu.sync_copy(table_hbm, table_v)
        pltpu.sync_copy(ids_hbm.at[pl.ds(base, _CHUNK)], ids_v)
        lanes = lax.iota(jnp.int32, _L)
        for g in range(_CHUNK // _L):
            ids16 = ids_v[pl.ds(g * _L, _L)]
            rowbase = ids16 * EMB_DIM
            rows = lanes + g * _L
            for c in range(EMB_DIM):
                vals = plsc.load_gather(table_v, [rowbase + c])
                cols = jnp.full((_L,), c, jnp.int32)
                plsc.store_scatter(out_v, [rows, cols], vals)
        pltpu.sync_copy(out_v, out_hbm.at[pl.ds(base, _CHUNK)])

    return _sc_gather


def kernel(input1, input2, embedding_table):
    ids = _edit_distance_ids(input1.T, input2.T)
    return _sc_gather_fn()(embedding_table.reshape(-1), ids)


# speedup vs baseline: 15.1580x; 1.3189x over previous
import functools

import jax
import jax.numpy as jnp
from jax import lax
from jax.experimental import pallas as pl
from jax.experimental.pallas import tpu as pltpu
from jax.experimental.pallas import tpu_sc as plsc

BATCH = 16384
SEQ = 20
EMB_SIZE = 512
EMB_DIM = 4

_BB = 2048
_BIG = 128.0


def _dp_body(a_ref, b_ref, out_ref):
    a = a_ref[...].astype(jnp.bfloat16)
    b = b_ref[...].astype(jnp.bfloat16)
    bb = a.shape[1]
    one = jnp.bfloat16(1)
    r = jnp.zeros((SEQ + 1, bb), jnp.bfloat16)
    for i in range(SEQ):
        cm1 = jnp.where(a[i:i + 1, :] != b, jnp.bfloat16(0),
                        jnp.bfloat16(-1))
        u = jnp.minimum(
            r + one,
            jnp.concatenate([jnp.full((1, bb), i + 1, jnp.bfloat16),
                             r[:SEQ, :] + cm1], axis=0))
        for s in (1, 2, 4, 8, 16):
            shifted = jnp.concatenate(
                [jnp.full((s, bb), _BIG, jnp.bfloat16), u[:SEQ + 1 - s, :]],
                axis=0)
            u = jnp.minimum(u, shifted)
        r = u
    dist = r[SEQ, :].astype(jnp.int32) + SEQ
    out_ref[...] = jnp.minimum(dist, EMB_SIZE - 1).reshape(1, bb)


def _edit_distance_ids(a_t, b_t):
    grid = BATCH // _BB
    out = pl.pallas_call(
        _dp_body,
        grid=(grid,),
        in_specs=[
            pl.BlockSpec((SEQ, _BB), lambda i: (0, i)),
            pl.BlockSpec((SEQ, _BB), lambda i: (0, i)),
        ],
        out_specs=pl.BlockSpec((1, _BB), lambda i: (0, i)),
        out_shape=jax.ShapeDtypeStruct((1, BATCH), jnp.int32),
    )(a_t, b_t)
    return out.reshape(BATCH)


_NC = 2
_NS = 16
_NW = _NC * _NS
_L = 16
_CHUNK = BATCH // _NW
_IDXW = 128
_DPAD = 16


@functools.cache
def _sc_gather_fn():
    mesh = plsc.VectorSubcoreMesh(
        core_axis_name="c", subcore_axis_name="s",
        num_cores=_NC, num_subcores=_NS)

    @functools.partial(
        pl.kernel,
        mesh=mesh,
        out_type=jax.ShapeDtypeStruct((BATCH // 128, EMB_DIM, 128), jnp.float32),
        scratch_types=[
            pltpu.VMEM((EMB_SIZE * EMB_DIM,), jnp.float32),
            pltpu.VMEM((_CHUNK,), jnp.int32),
            pltpu.VMEM((_CHUNK // 128, EMB_DIM, 128), jnp.float32),
        ],
        compiler_params=pltpu.CompilerParams(
            use_tc_tiling_on_sc=False, needs_layout_passes=False),
    )
    def _sc_gather(table_hbm, ids_hbm, out_hbm, table_v, ids_v, out_v):
        wid = lax.axis_index("s") * _NC + lax.axis_index("c")
        base = wid * _CHUNK
        pltpu.sync_copy(table_hbm, table_v)
        pltpu.sync_copy(ids_hbm.at[pl.ds(base, _CHUNK)], ids_v)
        for g in range(_CHUNK // _L):
            ids16 = ids_v[pl.ds(g * _L, _L)]
            rowbase = ids16 * EMB_DIM
            for c in range(EMB_DIM):
                vals = plsc.load_gather(table_v, [rowbase + c])
                out_v[g // 8, c, pl.ds((g % 8) * _L, _L)] = vals
        pltpu.sync_copy(
            out_v, out_hbm.at[pl.ds(wid * (_CHUNK // 128), _CHUNK // 128)])

    return _sc_gather


def kernel(input1, input2, embedding_table):
    ids = _edit_distance_ids(input1.T, input2.T)
    out3 = _sc_gather_fn()(embedding_table.reshape(-1), ids)
    return jnp.swapaxes(out3, 1, 2).reshape(BATCH, EMB_DIM)
